# Initial kernel scaffold; baseline (speedup 1.0000x reference)
import jax
import jax.numpy as jnp
from jax.experimental import pallas as pl


def kernel(out_1, out_2, xy_1, xy_2, nonmatch_2):
    raise NotImplementedError("write your pallas kernel here")


# speedup vs baseline: 1.1828x; 1.1828x over previous
import functools

import jax
import jax.numpy as jnp
from jax import lax
from jax.experimental import pallas as pl
from jax.experimental.pallas import tpu as pltpu
from jax.experimental.pallas import tpu_sc as plsc

_B, _C, _W, _H = 8, 96, 224, 224
_P = _W * _H
_N = 4096
_M = 8
_CG = 4
_CPG = _C // _CG
_EPS = 1e-7
_MARGIN = 0.5


def _sc_accumulate(out1v, out2v, i1p, idx2, idxn):
  mesh = plsc.VectorSubcoreMesh(core_axis_name="c", subcore_axis_name="s")

  @functools.partial(
      pl.kernel,
      mesh=mesh,
      compiler_params=pltpu.CompilerParams(
          use_tc_tiling_on_sc=False,
          needs_layout_passes=False,
      ),
      out_type=[
          jax.ShapeDtypeStruct((_CG, _B, _N), jnp.float32),
          jax.ShapeDtypeStruct((_CG, _B, _M * _N), jnp.float32),
      ],
      scratch_types=[
          pltpu.VMEM((_P,), jnp.float32),
          pltpu.VMEM((_N // 2,), jnp.int32),
          pltpu.VMEM((_N,), jnp.int32),
          pltpu.VMEM((_M * _N,), jnp.int32),
          pltpu.VMEM((_N,), jnp.float32),
          pltpu.VMEM((_N,), jnp.float32),
          pltpu.VMEM((_M * _N,), jnp.float32),
      ],
  )
  def k(o1_hbm, o2_hbm, i1p_hbm, i2_hbm, inn_hbm, accp_hbm, accn_hbm,
        plane_v, i1p_v, i2_v, inn_v, v1_v, ap_v, an_v):
    wid = lax.axis_index("s") * 2 + lax.axis_index("c")
    b = wid // _CG
    cg = wid % _CG

    pltpu.sync_copy(i1p_hbm.at[b], i1p_v)
    pltpu.sync_copy(i2_hbm.at[b], i2_v)
    pltpu.sync_copy(inn_hbm.at[b], inn_v)

    zeros = jnp.zeros((16,), jnp.float32)

    def zero_p(i, _):
      ap_v[pl.ds(i * 16, 16)] = zeros
      return 0

    lax.fori_loop(0, _N // 16, zero_p, 0)

    def zero_n(i, _):
      an_v[pl.ds(i * 16, 16)] = zeros
      return 0

    lax.fori_loop(0, _M * _N // 16, zero_n, 0)

    def channel(kk, _):
      c = cg * _CPG + kk
      pltpu.sync_copy(o1_hbm.at[b, c], plane_v)

      def g1(t, _):
        w = i1p_v[pl.ds(t * 16, 16)]
        lo = w & 0xFFFF
        hi = lax.shift_right_logical(w, 16)
        v1_v[pl.ds(t * 32, 16)] = plsc.load_gather(plane_v, [lo])
        v1_v[pl.ds(t * 32 + 16, 16)] = plsc.load_gather(plane_v, [hi])
        return 0

      lax.fori_loop(0, _N // 32, g1, 0)

      pltpu.sync_copy(o2_hbm.at[b, c], plane_v)

      def g2(t, _):
        off = t * 16
        v2 = plsc.load_gather(plane_v, [i2_v[pl.ds(off, 16)]])
        d = v1_v[pl.ds(off, 16)] - v2
        ap_v[pl.ds(off, 16)] = ap_v[pl.ds(off, 16)] + d * d
        return 0

      lax.fori_loop(0, _N // 16, g2, 0)

      def gneg(j, _):
        joff = j * 16
        w = plsc.load_gather(plane_v, [inn_v[pl.ds(joff, 16)]])
        d = v1_v[pl.ds((j % (_N // 16)) * 16, 16)] - w
        an_v[pl.ds(joff, 16)] = an_v[pl.ds(joff, 16)] + d * d
        return 0

      lax.fori_loop(0, _M * _N // 16, gneg, 0)
      return 0

    lax.fori_loop(0, _CPG, channel, 0)

    pltpu.sync_copy(ap_v, accp_hbm.at[cg, b])
    pltpu.sync_copy(an_v, accn_hbm.at[cg, b])

  return k(out1v, out2v, i1p, idx2, idxn)


def _final_kernel(ap_ref, an_ref, o_ref):
  p = ap_ref[0] + ap_ref[1] + ap_ref[2] + ap_ref[3]
  pos = jnp.sqrt(p + _EPS)
  nacc = an_ref[0] + an_ref[1] + an_ref[2] + an_ref[3]
  neg = jnp.sqrt(nacc + _EPS)
  total = jnp.float32(0.0)
  for b in range(_B):
    negm = jnp.sum(neg[b * _M:(b + 1) * _M], axis=0) * (1.0 / _M)
    terms = jnp.maximum(pos[b] - negm + _MARGIN, 0.0)
    total = total + jnp.sum(terms)
  o_ref[0, 0] = total / (_B * _N)


def kernel(out_1, out_2, xy_1, xy_2, nonmatch_2):
  out1v = out_1.reshape(_B, _C, _P)
  out2v = out_2.reshape(_B, _C, _P)

  xy_1 = xy_1.astype(jnp.int32)
  xy_2 = xy_2.astype(jnp.int32)
  nonmatch_2 = nonmatch_2.astype(jnp.int32)

  idx1 = xy_1[..., 0] * _H + xy_1[..., 1]
  idx2 = xy_2[..., 0] * _H + xy_2[..., 1]
  idxn = (nonmatch_2[..., 0] * _H + nonmatch_2[..., 1]).reshape(_B, _M * _N)

  r = idx1.reshape(_B, _N // 32, 2, 16)
  i1p = (r[:, :, 0, :] | (r[:, :, 1, :] << 16)).reshape(_B, _N // 2)

  accp, accn = _sc_accumulate(out1v, out2v, i1p, idx2, idxn)

  loss = pl.pallas_call(
      _final_kernel,
      out_shape=jax.ShapeDtypeStruct((1, 1), jnp.float32),
      out_specs=pl.BlockSpec(memory_space=pltpu.SMEM),
  )(accp, accn.reshape(_CG, _B * _M, _N))
  return loss[0, 0]


# speedup vs baseline: 2.2591x; 1.9099x over previous
import functools

import jax
import jax.numpy as jnp
from jax import lax
from jax.experimental import pallas as pl
from jax.experimental.pallas import tpu as pltpu
from jax.experimental.pallas import tpu_sc as plsc

_B, _C, _W, _H = 8, 96, 224, 224
_P = _W * _H
_N = 4096
_M = 8
_CG = 4
_CPG = _C // _CG
_EPS = 1e-7
_MARGIN = 0.5


def _sc_accumulate(out1v, out2v, i1p, idx2, idxn):
  mesh = plsc.VectorSubcoreMesh(core_axis_name="c", subcore_axis_name="s")

  @functools.partial(
      pl.kernel,
      mesh=mesh,
      compiler_params=pltpu.CompilerParams(
          use_tc_tiling_on_sc=False,
          needs_layout_passes=False,
      ),
      out_type=[
          jax.ShapeDtypeStruct((_CG, _B, _N), jnp.float32),
          jax.ShapeDtypeStruct((_CG, _B, _M * _N), jnp.float32),
      ],
      scratch_types=[
          pltpu.VMEM((_P,), jnp.float32),
          pltpu.VMEM((_N // 2,), jnp.int32),
          pltpu.VMEM((_N,), jnp.int32),
          pltpu.VMEM((_M * _N,), jnp.int32),
          pltpu.VMEM((_N,), jnp.float32),
          pltpu.VMEM((_N,), jnp.float32),
          pltpu.VMEM((_M * _N,), jnp.float32),
      ],
  )
  def k(o1_hbm, o2_hbm, i1p_hbm, i2_hbm, inn_hbm, accp_hbm, accn_hbm,
        plane_v, i1p_v, i2_v, inn_v, v1_v, ap_v, an_v):
    wid = lax.axis_index("s") * 2 + lax.axis_index("c")
    b = wid // _CG
    cg = wid % _CG

    pltpu.sync_copy(i1p_hbm.at[b], i1p_v)
    pltpu.sync_copy(i2_hbm.at[b], i2_v)
    pltpu.sync_copy(inn_hbm.at[b], inn_v)

    zeros = jnp.zeros((16,), jnp.float32)

    @plsc.parallel_loop(0, _N // 16, unroll=8)
    def zero_p(i):
      ap_v[pl.ds(i * 16, 16)] = zeros

    @plsc.parallel_loop(0, _M * _N // 16, unroll=8)
    def zero_n(i):
      an_v[pl.ds(i * 16, 16)] = zeros

    def channel(kk, _):
      c = cg * _CPG + kk
      pltpu.sync_copy(o1_hbm.at[b, c], plane_v)

      @plsc.parallel_loop(0, _N // 32, unroll=4)
      def g1(t):
        w = i1p_v[pl.ds(t * 16, 16)]
        lo = w & 0xFFFF
        hi = lax.shift_right_logical(w, 16)
        v1_v[pl.ds(t * 32, 16)] = plsc.load_gather(plane_v, [lo])
        v1_v[pl.ds(t * 32 + 16, 16)] = plsc.load_gather(plane_v, [hi])

      pltpu.sync_copy(o2_hbm.at[b, c], plane_v)

      @plsc.parallel_loop(0, _N // 16, unroll=4)
      def g2(t):
        off = t * 16
        v2 = plsc.load_gather(plane_v, [i2_v[pl.ds(off, 16)]])
        d = v1_v[pl.ds(off, 16)] - v2
        plsc.addupdate(ap_v.at[pl.ds(off, 16)], d * d)

      @plsc.parallel_loop(0, _N // 16, unroll=2)
      def gneg(t):
        off = t * 16
        v1 = v1_v[pl.ds(off, 16)]
        for mm in range(_M):
          joff = mm * _N + off
          w = plsc.load_gather(plane_v, [inn_v[pl.ds(joff, 16)]])
          d = v1 - w
          plsc.addupdate(an_v.at[pl.ds(joff, 16)], d * d)

      return 0

    lax.fori_loop(0, _CPG, channel, 0)

    pltpu.sync_copy(ap_v, accp_hbm.at[cg, b])
    pltpu.sync_copy(an_v, accn_hbm.at[cg, b])

  return k(out1v, out2v, i1p, idx2, idxn)


def _final_kernel(ap_ref, an_ref, o_ref):
  p = ap_ref[0] + ap_ref[1] + ap_ref[2] + ap_ref[3]
  pos = jnp.sqrt(p + _EPS)
  nacc = an_ref[0] + an_ref[1] + an_ref[2] + an_ref[3]
  neg = jnp.sqrt(nacc + _EPS)
  total = jnp.float32(0.0)
  for b in range(_B):
    negm = jnp.sum(neg[b * _M:(b + 1) * _M], axis=0) * (1.0 / _M)
    terms = jnp.maximum(pos[b] - negm + _MARGIN, 0.0)
    total = total + jnp.sum(terms)
  o_ref[0, 0] = total / (_B * _N)


def kernel(out_1, out_2, xy_1, xy_2, nonmatch_2):
  out1v = out_1.reshape(_B, _C, _P)
  out2v = out_2.reshape(_B, _C, _P)

  xy_1 = xy_1.astype(jnp.int32)
  xy_2 = xy_2.astype(jnp.int32)
  nonmatch_2 = nonmatch_2.astype(jnp.int32)

  idx1 = xy_1[..., 0] * _H + xy_1[..., 1]
  idx2 = xy_2[..., 0] * _H + xy_2[..., 1]
  idxn = (nonmatch_2[..., 0] * _H + nonmatch_2[..., 1]).reshape(_B, _M * _N)

  r = idx1.reshape(_B, _N // 32, 2, 16)
  i1p = (r[:, :, 0, :] | (r[:, :, 1, :] << 16)).reshape(_B, _N // 2)

  accp, accn = _sc_accumulate(out1v, out2v, i1p, idx2, idxn)

  loss = pl.pallas_call(
      _final_kernel,
      out_shape=jax.ShapeDtypeStruct((1, 1), jnp.float32),
      out_specs=pl.BlockSpec(memory_space=pltpu.SMEM),
  )(accp, accn.reshape(_CG, _B * _M, _N))
  return loss[0, 0]


# speedup vs baseline: 2.4855x; 1.1002x over previous
import functools

import jax
import jax.numpy as jnp
from jax import lax
from jax.experimental import pallas as pl
from jax.experimental.pallas import tpu as pltpu
from jax.experimental.pallas import tpu_sc as plsc

_B, _C, _W, _H = 8, 96, 224, 224
_P = _W * _H
_N = 4096
_M = 8
_CG = 4
_CPG = _C // _CG
_EPS = 1e-7
_MARGIN = 0.5


def _sc_accumulate(out1f, out2v, idx1, idx2, innp):
  mesh = plsc.VectorSubcoreMesh(core_axis_name="c", subcore_axis_name="s")

  @functools.partial(
      pl.kernel,
      mesh=mesh,
      compiler_params=pltpu.CompilerParams(
          use_tc_tiling_on_sc=False,
          needs_layout_passes=False,
      ),
      out_type=[
          jax.ShapeDtypeStruct((_CG, _B, _N), jnp.float32),
          jax.ShapeDtypeStruct((_CG, _B, _M * _N), jnp.float32),
      ],
      scratch_types=[
          pltpu.VMEM((_P,), jnp.float32),
          pltpu.VMEM((_N,), jnp.int32),
          pltpu.VMEM((_N,), jnp.int32),
          pltpu.VMEM((_N,), jnp.int32),
          pltpu.VMEM((_N,), jnp.int32),
          pltpu.VMEM((_M // 2 * _N,), jnp.int32),
          pltpu.VMEM((_N,), jnp.float32),
          pltpu.VMEM((_N,), jnp.float32),
          pltpu.VMEM((_N,), jnp.float32),
          pltpu.VMEM((_M * _N,), jnp.float32),
          pltpu.SemaphoreType.DMA,
          pltpu.SemaphoreType.DMA,
          pltpu.SemaphoreType.DMA,
      ],
  )
  def k(o1f_hbm, o2_hbm, i1_hbm, i2_hbm, innp_hbm, accp_hbm, accn_hbm,
        plane_v, i1_v, ia_a, ia_b, i2_v, innp_v, v1_a, v1_b, ap_v, an_v,
        semp, sga, sgb):
    wid = lax.axis_index("s") * 2 + lax.axis_index("c")
    b = wid // _CG
    cg = wid % _CG
    c0 = cg * _CPG

    pltpu.sync_copy(i1_hbm.at[b], i1_v)
    pltpu.sync_copy(i2_hbm.at[b], i2_v)
    pltpu.sync_copy(innp_hbm.at[b], innp_v)

    zeros = jnp.zeros((16,), jnp.float32)

    @plsc.parallel_loop(0, _N // 16, unroll=8)
    def zero_p(i):
      ap_v[pl.ds(i * 16, 16)] = zeros

    @plsc.parallel_loop(0, _M * _N // 16, unroll=8)
    def zero_n(i):
      an_v[pl.ds(i * 16, 16)] = zeros

    def fill_ia(ia_ref, c):
      base = (b * _C + c) * _P

      @plsc.parallel_loop(0, _N // 16, unroll=8)
      def _(t):
        ia_ref[pl.ds(t * 16, 16)] = i1_v[pl.ds(t * 16, 16)] + base

    fill_ia(ia_a, c0)
    pltpu.async_copy(o1f_hbm.at[ia_a], v1_a, sga)
    pltpu.async_copy(o2_hbm.at[b, c0], plane_v, semp)

    def body(kk, ia_cur, v1_cur, sem_cur, ia_nxt, v1_nxt, sem_nxt):
      c = c0 + kk
      cnx = c0 + jnp.minimum(kk + 1, _CPG - 1)
      fill_ia(ia_nxt, cnx)
      pltpu.async_copy(o1f_hbm.at[ia_nxt], v1_nxt, sem_nxt)
      pltpu.make_async_copy(o1f_hbm.at[ia_cur], v1_cur, sem_cur).wait()
      pltpu.make_async_copy(o2_hbm.at[b, c], plane_v, semp).wait()

      @plsc.parallel_loop(0, _N // 16, unroll=2)
      def g(t):
        off = t * 16
        v1 = v1_cur[pl.ds(off, 16)]
        v2 = plsc.load_gather(plane_v, [i2_v[pl.ds(off, 16)]])
        d = v1 - v2
        plsc.addupdate(ap_v.at[pl.ds(off, 16)], d * d)
        for q in range(_M // 2):
          w16 = innp_v[pl.ds(q * _N + off, 16)]
          wlo = plsc.load_gather(plane_v, [w16 & 0xFFFF])
          whi = plsc.load_gather(plane_v, [lax.shift_right_logical(w16, 16)])
          dlo = v1 - wlo
          dhi = v1 - whi
          plsc.addupdate(an_v.at[pl.ds(2 * q * _N + off, 16)], dlo * dlo)
          plsc.addupdate(an_v.at[pl.ds((2 * q + 1) * _N + off, 16)], dhi * dhi)

      pltpu.async_copy(o2_hbm.at[b, cnx], plane_v, semp)

    def pair(j, _):
      body(2 * j, ia_a, v1_a, sga, ia_b, v1_b, sgb)
      body(2 * j + 1, ia_b, v1_b, sgb, ia_a, v1_a, sga)
      return 0

    lax.fori_loop(0, _CPG // 2, pair, 0)

    pltpu.make_async_copy(o1f_hbm.at[ia_a], v1_a, sga).wait()
    pltpu.make_async_copy(o2_hbm.at[b, c0 + _CPG - 1], plane_v, semp).wait()

    pltpu.sync_copy(ap_v, accp_hbm.at[cg, b])
    pltpu.sync_copy(an_v, accn_hbm.at[cg, b])

  return k(out1f, out2v, idx1, idx2, innp)


def _final_kernel(ap_ref, an_ref, o_ref):
  p = ap_ref[0] + ap_ref[1] + ap_ref[2] + ap_ref[3]
  pos = jnp.sqrt(p + _EPS)
  nacc = an_ref[0] + an_ref[1] + an_ref[2] + an_ref[3]
  neg = jnp.sqrt(nacc + _EPS)
  total = jnp.float32(0.0)
  for b in range(_B):
    negm = jnp.sum(neg[b * _M:(b + 1) * _M], axis=0) * (1.0 / _M)
    terms = jnp.maximum(pos[b] - negm + _MARGIN, 0.0)
    total = total + jnp.sum(terms)
  o_ref[0, 0] = total / (_B * _N)


def kernel(out_1, out_2, xy_1, xy_2, nonmatch_2):
  out1f = out_1.reshape(_B * _C * _P)
  out2v = out_2.reshape(_B, _C, _P)

  xy_1 = xy_1.astype(jnp.int32)
  xy_2 = xy_2.astype(jnp.int32)
  nonmatch_2 = nonmatch_2.astype(jnp.int32)

  idx1 = xy_1[..., 0] * _H + xy_1[..., 1]
  idx2 = xy_2[..., 0] * _H + xy_2[..., 1]
  idxn = nonmatch_2[..., 0] * _H + nonmatch_2[..., 1]

  innp = (idxn[:, 0::2, :] | (idxn[:, 1::2, :] << 16)).reshape(_B, _M // 2 * _N)

  accp, accn = _sc_accumulate(out1f, out2v, idx1, idx2, innp)

  loss = pl.pallas_call(
      _final_kernel,
      out_shape=jax.ShapeDtypeStruct((1, 1), jnp.float32),
      out_specs=pl.BlockSpec(memory_space=pltpu.SMEM),
  )(accp, accn.reshape(_CG, _B * _M, _N))
  return loss[0, 0]
